# baseline (speedup 1.0000x reference)
import functools

import jax
import jax.numpy as jnp
from jax import lax
from jax.experimental import pallas as pl
from jax.experimental.pallas import tpu as pltpu
from jax.experimental.pallas import tpu_sc as plsc

NN = 10000
NP = 10240
EE = 320000
EPAD = 327680
FX = 128
FE = 16
FO = 128
EPS = 1e-5
SLOPE = 0.1

CH = 128
NCHUNK = EPAD // CH
TPS = NP // 16


def _lrelu(v):
    return jnp.where(v >= 0.0, v, SLOPE * v)


def _sc_gather(p_pad, row):
    info = plsc.get_sparse_core_info()
    nc, ns = info.num_cores, info.num_subcores
    nw = nc * ns
    steps = NCHUNK // nw
    nbuf = 2
    groups = steps // nbuf

    @functools.partial(
        pl.kernel,
        mesh=plsc.VectorSubcoreMesh(core_axis_name="c", subcore_axis_name="s"),
        out_type=jax.ShapeDtypeStruct((EPAD, FO), jnp.float32),
        scratch_types=[
            pltpu.VMEM((steps, CH), jnp.int32),
            pltpu.VMEM((nbuf, CH, FO), jnp.float32),
        ] + [pltpu.SemaphoreType.DMA] * (2 * nbuf),
    )
    def k(p_hbm, row_hbm, out_hbm, idx_all, rows_v, *sems):
        sem_g = sems[:nbuf]
        sem_w = sems[nbuf:]
        cid = lax.axis_index("c")
        sid = lax.axis_index("s")
        wid = sid * nc + cid
        c0 = wid * steps

        pltpu.sync_copy(row_hbm.at[pl.ds(c0, steps)], idx_all)

        def gstart(u, t):
            pltpu.async_copy(p_hbm.at[idx_all.at[t]], rows_v.at[u], sem_g[u])

        def gwait(u, t):
            pltpu.make_async_copy(
                p_hbm.at[idx_all.at[t]], rows_v.at[u], sem_g[u]).wait()

        def wstart(u, t):
            pltpu.async_copy(rows_v.at[u],
                             out_hbm.at[pl.ds((c0 + t) * CH, CH)], sem_w[u])

        def wwait(u):
            pltpu.make_async_copy(
                rows_v.at[u], out_hbm.at[pl.ds(0, CH)], sem_w[u]).wait()

        def body(t, carry):
            gstart(0, t)
            gwait(0, t)
            pltpu.sync_copy(rows_v.at[0], out_hbm.at[pl.ds((c0 + t) * CH, CH)])
            return carry

        lax.fori_loop(0, steps, body, 0)

    return k(p_pad, row)


def _sc_scatter(z2, col):
    info = plsc.get_sparse_core_info()
    nc, ns = info.num_cores, info.num_subcores
    nw = nc * ns
    steps = NCHUNK // nw
    nbuf = 2
    groups = steps // nbuf

    @functools.partial(
        pl.kernel,
        mesh=plsc.VectorSubcoreMesh(core_axis_name="c", subcore_axis_name="s"),
        out_type=(
            jax.ShapeDtypeStruct((2, NP, FO), jnp.float32),
            jax.ShapeDtypeStruct((2, NP), jnp.float32),
        ),
        scratch_types=[
            pltpu.VMEM((steps, CH), jnp.int32),
            pltpu.VMEM((nbuf, CH, FO), jnp.float32),
            pltpu.VMEM((TPS,), jnp.float32),
            pltpu.VMEM((CH,), jnp.float32),
            pltpu.VMEM_SHARED((NP, FO), jnp.float32),
            pltpu.VMEM_SHARED((NP,), jnp.float32),
        ] + [pltpu.SemaphoreType.DMA] * (3 * nbuf),
    )
    def k(z2_hbm, col_hbm, sum_hbm, cnt_hbm, idx_all, rows_v, zcnt,
          ones_v, acc, acc_cnt, *sems):
        sem_z = sems[:nbuf]
        sem_s = sems[nbuf:2 * nbuf]
        sem_c = sems[2 * nbuf:]
        cid = lax.axis_index("c")
        sid = lax.axis_index("s")
        wid = sid * nc + cid
        zf = jnp.zeros((16,), jnp.float32)
        of = jnp.ones((16,), jnp.float32)

        def zero_blk(i, carry):
            rows_v[0, i // 8, pl.ds((i % 8) * 16, 16)] = zf
            return carry

        lax.fori_loop(0, CH * FO // 16, zero_blk, 0)

        def zero_cnt(i, carry):
            zcnt[pl.ds(i * 16, 16)] = zf
            return carry

        lax.fori_loop(0, TPS // 16, zero_cnt, 0)
        for i in range(CH // 16):
            ones_v[pl.ds(i * 16, 16)] = of

        for i in range(TPS // CH):
            pltpu.sync_copy(rows_v.at[0], acc.at[pl.ds(sid * TPS + i * CH, CH)])
        pltpu.sync_copy(zcnt, acc_cnt.at[pl.ds(sid * TPS, TPS)])

        c0 = wid * steps
        pltpu.sync_copy(col_hbm.at[pl.ds(c0, steps)], idx_all)
        plsc.subcore_barrier()

        def zstart(u, t):
            pltpu.async_copy(z2_hbm.at[pl.ds((c0 + t) * CH, CH)],
                             rows_v.at[u], sem_z[u])

        def zwait(u):
            pltpu.make_async_copy(
                z2_hbm.at[pl.ds(0, CH)], rows_v.at[u], sem_z[u]).wait()

        def sstart(u, t):
            pltpu.async_copy(rows_v.at[u], acc.at[idx_all.at[t]], sem_s[u],
                             add=True)
            pltpu.async_copy(ones_v, acc_cnt.at[idx_all.at[t]], sem_c[u],
                             add=True)

        def swait(u, t):
            pltpu.make_async_copy(
                rows_v.at[u], acc.at[idx_all.at[t]], sem_s[u]).wait()
            pltpu.make_async_copy(
                ones_v, acc_cnt.at[idx_all.at[t]], sem_c[u]).wait()

        for u in range(nbuf):
            zstart(u, u)

        def body(j2, carry):
            for u in range(nbuf):
                t = nbuf * j2 + u
                zwait(u)
                sstart(u, t)
                swait(u, t)
                zstart(u, t + nbuf)
            return carry

        lax.fori_loop(0, groups - 1, body, 0)
        for u in range(nbuf):
            t = nbuf * (groups - 1) + u
            zwait(u)
            sstart(u, t)
            swait(u, t)
        plsc.subcore_barrier()

        pltpu.sync_copy(acc.at[pl.ds(sid * TPS, TPS)],
                        sum_hbm.at[cid].at[pl.ds(sid * TPS, TPS)])
        pltpu.sync_copy(acc_cnt.at[pl.ds(sid * TPS, TPS)],
                        cnt_hbm.at[cid].at[pl.ds(sid * TPS, TPS)])

    return k(z2, col)


def _tc_pre(x_pad, wx, b):
    br = 1280

    def body(x_ref, w_ref, b_ref, o_ref):
        o_ref[...] = (
            jnp.dot(x_ref[...], w_ref[...], preferred_element_type=jnp.float32)
            + b_ref[...]
        )

    return pl.pallas_call(
        body,
        grid=(NP // br,),
        in_specs=[
            pl.BlockSpec((br, FX), lambda i: (i, 0)),
            pl.BlockSpec((FX, FO), lambda i: (0, 0)),
            pl.BlockSpec((1, FO), lambda i: (0, 0)),
        ],
        out_specs=pl.BlockSpec((br, FO), lambda i: (i, 0)),
        out_shape=jax.ShapeDtypeStruct((NP, FO), jnp.float32),
    )(x_pad, wx, b.reshape(1, FO))


def _tc_edge(gathered, ea, wae, wb, bbias):
    be = 4096

    def body(g_ref, e_ref, wae_ref, wb_ref, b_ref, o_ref):
        h = g_ref[...] + jnp.dot(
            e_ref[...], wae_ref[...], preferred_element_type=jnp.float32
        )
        h = _lrelu(h)
        h = jnp.dot(h, wb_ref[...], preferred_element_type=jnp.float32) + b_ref[...]
        o_ref[...] = _lrelu(h)

    return pl.pallas_call(
        body,
        grid=(EPAD // be,),
        in_specs=[
            pl.BlockSpec((be, FO), lambda i: (i, 0)),
            pl.BlockSpec((be, FE), lambda i: (i, 0)),
            pl.BlockSpec((FE, FO), lambda i: (0, 0)),
            pl.BlockSpec((FO, FO), lambda i: (0, 0)),
            pl.BlockSpec((1, FO), lambda i: (0, 0)),
        ],
        out_specs=pl.BlockSpec((be, FO), lambda i: (i, 0)),
        out_shape=jax.ShapeDtypeStruct((EPAD, FO), jnp.float32),
    )(gathered, ea, wae, wb, bbias.reshape(1, FO))


def _tc_node(s_part, c_part, x_pad, wc, bc, wax, wag, ba, wb, bb2, wc2, bc2):
    br = 1280
    c3 = c_part.reshape(2, NP, 1)

    def body(s_ref, c_ref, x_ref, wc_ref, bc_ref, wax_ref, wag_ref, ba_ref,
             wb_ref, bb_ref, wc2_ref, bc2_ref, o_ref):
        s = s_ref[0] + s_ref[1]
        cnt = c_ref[0] + c_ref[1]
        sums = (
            jnp.dot(s, wc_ref[...], preferred_element_type=jnp.float32)
            + cnt * bc_ref[...]
        )
        agg = sums / jnp.maximum(cnt, 1.0)
        h = (
            jnp.dot(x_ref[...], wax_ref[...], preferred_element_type=jnp.float32)
            + jnp.dot(agg, wag_ref[...], preferred_element_type=jnp.float32)
            + ba_ref[...]
        )
        h = _lrelu(h)
        h = jnp.dot(h, wb_ref[...], preferred_element_type=jnp.float32) + bb_ref[...]
        h = _lrelu(h)
        o_ref[...] = (
            jnp.dot(h, wc2_ref[...], preferred_element_type=jnp.float32)
            + bc2_ref[...]
        )

    return pl.pallas_call(
        body,
        grid=(NP // br,),
        in_specs=[
            pl.BlockSpec((2, br, FO), lambda i: (0, i, 0)),
            pl.BlockSpec((2, br, 1), lambda i: (0, i, 0)),
            pl.BlockSpec((br, FX), lambda i: (i, 0)),
            pl.BlockSpec((FO, FO), lambda i: (0, 0)),
            pl.BlockSpec((1, FO), lambda i: (0, 0)),
            pl.BlockSpec((FX, FO), lambda i: (0, 0)),
            pl.BlockSpec((FO, FO), lambda i: (0, 0)),
            pl.BlockSpec((1, FO), lambda i: (0, 0)),
            pl.BlockSpec((FO, FO), lambda i: (0, 0)),
            pl.BlockSpec((1, FO), lambda i: (0, 0)),
            pl.BlockSpec((FO, FO), lambda i: (0, 0)),
            pl.BlockSpec((1, FO), lambda i: (0, 0)),
        ],
        out_specs=pl.BlockSpec((br, FO), lambda i: (i, 0)),
        out_shape=jax.ShapeDtypeStruct((NP, FO), jnp.float32),
    )(s_part, c3, x_pad, wc, bc.reshape(1, FO), wax, wag, ba.reshape(1, FO),
      wb, bb2.reshape(1, FO), wc2, bc2.reshape(1, FO))


def _fold(g, bb_, rm, rv, w, lb):
    s = g * lax.rsqrt(rv + EPS)
    t = bb_ - rm * s
    return w * s[:, None], t @ w + lb


def kernel(x, edge_index, edge_attr, u, batch, g1a, bb1a, rm1a, rv1a, w1a,
           lb1a, g1b, bb1b, rm1b, rv1b, w1b, lb1b, g1c, bb1c, rm1c, rv1c,
           w1c, lb1c, g2a, bb2a, rm2a, rv2a, w2a, lb2a, g2b, bb2b, rm2b,
           rv2b, w2b, lb2b, g2c, bb2c, rm2c, rv2c, w2c, lb2c):
    w1a_f, b1a_f = _fold(g1a, bb1a, rm1a, rv1a, w1a, lb1a)
    w1b_f, b1b_f = _fold(g1b, bb1b, rm1b, rv1b, w1b, lb1b)
    w1c_f, b1c_f = _fold(g1c, bb1c, rm1c, rv1c, w1c, lb1c)
    w2a_f, b2a_f = _fold(g2a, bb2a, rm2a, rv2a, w2a, lb2a)
    w2b_f, b2b_f = _fold(g2b, bb2b, rm2b, rv2b, w2b, lb2b)
    w2c_f, b2c_f = _fold(g2c, bb2c, rm2c, rv2c, w2c, lb2c)
    w1ax, w1ae = w1a_f[:FX], w1a_f[FX:]
    w2ax, w2ag = w2a_f[:FX], w2a_f[FX:]

    ep = EPAD - EE
    row = jnp.concatenate(
        [edge_index[0].astype(jnp.int32), jnp.zeros((ep,), jnp.int32)])
    col = jnp.concatenate(
        [edge_index[1].astype(jnp.int32), jnp.full((ep,), NN, jnp.int32)])
    ea_pad = jnp.pad(edge_attr, ((0, ep), (0, 0)))
    x_pad = jnp.pad(x, ((0, NP - NN), (0, 0)))

    p = _tc_pre(x_pad, w1ax, b1a_f)
    gathered = _sc_gather(p, row.reshape(NCHUNK, CH))
    z2 = _tc_edge(gathered, ea_pad, w1ae, w1b_f, b1b_f)
    s_part, c_part = _sc_scatter(z2, col.reshape(NCHUNK, CH))
    out = _tc_node(s_part, c_part, x_pad, w1c_f, b1c_f, w2ax, w2ag, b2a_f,
                   w2b_f, b2b_f, w2c_f, b2c_f)
    return out[:NN]


# speedup vs baseline: 1.1801x; 1.1801x over previous
import functools

import jax
import jax.numpy as jnp
from jax import lax
from jax.experimental import pallas as pl
from jax.experimental.pallas import tpu as pltpu
from jax.experimental.pallas import tpu_sc as plsc

NN = 10000
NP = 10240
EE = 320000
EPAD = 327680
FX = 128
FE = 16
FO = 128
EPS = 1e-5
SLOPE = 0.1

CH = 128
NCHUNK = EPAD // CH
TPS = NP // 16


def _lrelu(v):
    return jnp.where(v >= 0.0, v, SLOPE * v)


def _sc_gather(p_pad, row):
    info = plsc.get_sparse_core_info()
    nc, ns = info.num_cores, info.num_subcores
    nw = nc * ns
    steps = NCHUNK // nw
    nbuf = 2
    groups = steps // nbuf

    @functools.partial(
        pl.kernel,
        mesh=plsc.VectorSubcoreMesh(core_axis_name="c", subcore_axis_name="s"),
        out_type=jax.ShapeDtypeStruct((EPAD, FO), jnp.float32),
        scratch_types=[
            pltpu.VMEM((steps, CH), jnp.int32),
            pltpu.VMEM((nbuf, CH, FO), jnp.float32),
        ] + [pltpu.SemaphoreType.DMA] * (2 * nbuf),
    )
    def k(p_hbm, row_hbm, out_hbm, idx_all, rows_v, *sems):
        sem_g = sems[:nbuf]
        sem_w = sems[nbuf:]
        cid = lax.axis_index("c")
        sid = lax.axis_index("s")
        wid = sid * nc + cid

        def body(t, carry):
            c = wid + t * nw
            pltpu.sync_copy(row_hbm.at[c], idx_all.at[0])
            pltpu.async_copy(p_hbm.at[idx_all.at[0]], rows_v.at[0],
                             sem_g[0]).wait()
            pltpu.sync_copy(rows_v.at[0], out_hbm.at[pl.ds(c * CH, CH)])
            return carry

        lax.fori_loop(0, steps, body, 0)

    return k(p_pad, row)


def _sc_scatter(z2, col):
    info = plsc.get_sparse_core_info()
    nc, ns = info.num_cores, info.num_subcores
    nw = nc * ns
    steps = NCHUNK // nw
    nbuf = 2
    groups = steps // nbuf

    @functools.partial(
        pl.kernel,
        mesh=plsc.VectorSubcoreMesh(core_axis_name="c", subcore_axis_name="s"),
        out_type=(
            jax.ShapeDtypeStruct((2, NP, FO), jnp.float32),
            jax.ShapeDtypeStruct((2, NP), jnp.float32),
        ),
        scratch_types=[
            pltpu.VMEM((steps, CH), jnp.int32),
            pltpu.VMEM((nbuf, CH, FO), jnp.float32),
            pltpu.VMEM((TPS,), jnp.float32),
            pltpu.VMEM((CH,), jnp.float32),
            pltpu.VMEM_SHARED((NP, FO), jnp.float32),
            pltpu.VMEM_SHARED((NP,), jnp.float32),
        ] + [pltpu.SemaphoreType.DMA] * (3 * nbuf),
    )
    def k(z2_hbm, col_hbm, sum_hbm, cnt_hbm, idx_all, rows_v, zcnt,
          ones_v, acc, acc_cnt, *sems):
        sem_z = sems[:nbuf]
        sem_s = sems[nbuf:2 * nbuf]
        sem_c = sems[2 * nbuf:]
        cid = lax.axis_index("c")
        sid = lax.axis_index("s")
        wid = sid * nc + cid
        zf = jnp.zeros((16,), jnp.float32)
        of = jnp.ones((16,), jnp.float32)

        def zero_blk(i, carry):
            rows_v[0, i // 8, pl.ds((i % 8) * 16, 16)] = zf
            return carry

        lax.fori_loop(0, CH * FO // 16, zero_blk, 0)

        def zero_cnt(i, carry):
            zcnt[pl.ds(i * 16, 16)] = zf
            return carry

        lax.fori_loop(0, TPS // 16, zero_cnt, 0)
        for i in range(CH // 16):
            ones_v[pl.ds(i * 16, 16)] = of

        for i in range(TPS // CH):
            pltpu.sync_copy(rows_v.at[0], acc.at[pl.ds(sid * TPS + i * CH, CH)])
        pltpu.sync_copy(zcnt, acc_cnt.at[pl.ds(sid * TPS, TPS)])

        c0 = wid * steps
        pltpu.sync_copy(col_hbm.at[pl.ds(c0, steps)], idx_all)
        plsc.subcore_barrier()

        def zstart(u, t):
            pltpu.async_copy(z2_hbm.at[pl.ds((c0 + t) * CH, CH)],
                             rows_v.at[u], sem_z[u])

        def zwait(u):
            pltpu.make_async_copy(
                z2_hbm.at[pl.ds(0, CH)], rows_v.at[u], sem_z[u]).wait()

        def sstart(u, t):
            pltpu.async_copy(rows_v.at[u], acc.at[idx_all.at[t]], sem_s[u],
                             add=True)
            pltpu.async_copy(ones_v, acc_cnt.at[idx_all.at[t]], sem_c[u],
                             add=True)

        def swait(u, t):
            pltpu.make_async_copy(
                rows_v.at[u], acc.at[idx_all.at[t]], sem_s[u]).wait()
            pltpu.make_async_copy(
                ones_v, acc_cnt.at[idx_all.at[t]], sem_c[u]).wait()

        for u in range(nbuf):
            zstart(u, u)

        def body(j2, carry):
            for u in range(nbuf):
                t = nbuf * j2 + u
                zwait(u)
                sstart(u, t)
                swait(u, t)
                zstart(u, t + nbuf)
            return carry

        lax.fori_loop(0, groups - 1, body, 0)
        for u in range(nbuf):
            t = nbuf * (groups - 1) + u
            zwait(u)
            sstart(u, t)
            swait(u, t)
        plsc.subcore_barrier()

        pltpu.sync_copy(acc.at[pl.ds(sid * TPS, TPS)],
                        sum_hbm.at[cid].at[pl.ds(sid * TPS, TPS)])
        pltpu.sync_copy(acc_cnt.at[pl.ds(sid * TPS, TPS)],
                        cnt_hbm.at[cid].at[pl.ds(sid * TPS, TPS)])

    return k(z2, col)


def _tc_pre(x_pad, wx, b):
    br = 1280

    def body(x_ref, w_ref, b_ref, o_ref):
        o_ref[...] = (
            jnp.dot(x_ref[...], w_ref[...], preferred_element_type=jnp.float32)
            + b_ref[...]
        )

    return pl.pallas_call(
        body,
        grid=(NP // br,),
        in_specs=[
            pl.BlockSpec((br, FX), lambda i: (i, 0)),
            pl.BlockSpec((FX, FO), lambda i: (0, 0)),
            pl.BlockSpec((1, FO), lambda i: (0, 0)),
        ],
        out_specs=pl.BlockSpec((br, FO), lambda i: (i, 0)),
        out_shape=jax.ShapeDtypeStruct((NP, FO), jnp.float32),
    )(x_pad, wx, b.reshape(1, FO))


def _tc_edge(gathered, ea, wae, wb, bbias):
    be = 4096

    def body(g_ref, e_ref, wae_ref, wb_ref, b_ref, o_ref):
        h = g_ref[...] + jnp.dot(
            e_ref[...], wae_ref[...], preferred_element_type=jnp.float32
        )
        h = _lrelu(h)
        h = jnp.dot(h, wb_ref[...], preferred_element_type=jnp.float32) + b_ref[...]
        o_ref[...] = _lrelu(h)

    return pl.pallas_call(
        body,
        grid=(EPAD // be,),
        in_specs=[
            pl.BlockSpec((be, FO), lambda i: (i, 0)),
            pl.BlockSpec((be, FE), lambda i: (i, 0)),
            pl.BlockSpec((FE, FO), lambda i: (0, 0)),
            pl.BlockSpec((FO, FO), lambda i: (0, 0)),
            pl.BlockSpec((1, FO), lambda i: (0, 0)),
        ],
        out_specs=pl.BlockSpec((be, FO), lambda i: (i, 0)),
        out_shape=jax.ShapeDtypeStruct((EPAD, FO), jnp.float32),
    )(gathered, ea, wae, wb, bbias.reshape(1, FO))


def _tc_node(s_part, c_part, x_pad, wc, bc, wax, wag, ba, wb, bb2, wc2, bc2):
    br = 1280
    c3 = c_part.reshape(2, NP, 1)

    def body(s_ref, c_ref, x_ref, wc_ref, bc_ref, wax_ref, wag_ref, ba_ref,
             wb_ref, bb_ref, wc2_ref, bc2_ref, o_ref):
        s = s_ref[0] + s_ref[1]
        cnt = c_ref[0] + c_ref[1]
        sums = (
            jnp.dot(s, wc_ref[...], preferred_element_type=jnp.float32)
            + cnt * bc_ref[...]
        )
        agg = sums / jnp.maximum(cnt, 1.0)
        h = (
            jnp.dot(x_ref[...], wax_ref[...], preferred_element_type=jnp.float32)
            + jnp.dot(agg, wag_ref[...], preferred_element_type=jnp.float32)
            + ba_ref[...]
        )
        h = _lrelu(h)
        h = jnp.dot(h, wb_ref[...], preferred_element_type=jnp.float32) + bb_ref[...]
        h = _lrelu(h)
        o_ref[...] = (
            jnp.dot(h, wc2_ref[...], preferred_element_type=jnp.float32)
            + bc2_ref[...]
        )

    return pl.pallas_call(
        body,
        grid=(NP // br,),
        in_specs=[
            pl.BlockSpec((2, br, FO), lambda i: (0, i, 0)),
            pl.BlockSpec((2, br, 1), lambda i: (0, i, 0)),
            pl.BlockSpec((br, FX), lambda i: (i, 0)),
            pl.BlockSpec((FO, FO), lambda i: (0, 0)),
            pl.BlockSpec((1, FO), lambda i: (0, 0)),
            pl.BlockSpec((FX, FO), lambda i: (0, 0)),
            pl.BlockSpec((FO, FO), lambda i: (0, 0)),
            pl.BlockSpec((1, FO), lambda i: (0, 0)),
            pl.BlockSpec((FO, FO), lambda i: (0, 0)),
            pl.BlockSpec((1, FO), lambda i: (0, 0)),
            pl.BlockSpec((FO, FO), lambda i: (0, 0)),
            pl.BlockSpec((1, FO), lambda i: (0, 0)),
        ],
        out_specs=pl.BlockSpec((br, FO), lambda i: (i, 0)),
        out_shape=jax.ShapeDtypeStruct((NP, FO), jnp.float32),
    )(s_part, c3, x_pad, wc, bc.reshape(1, FO), wax, wag, ba.reshape(1, FO),
      wb, bb2.reshape(1, FO), wc2, bc2.reshape(1, FO))


def _fold(g, bb_, rm, rv, w, lb):
    s = g * lax.rsqrt(rv + EPS)
    t = bb_ - rm * s
    return w * s[:, None], t @ w + lb


def kernel(x, edge_index, edge_attr, u, batch, g1a, bb1a, rm1a, rv1a, w1a,
           lb1a, g1b, bb1b, rm1b, rv1b, w1b, lb1b, g1c, bb1c, rm1c, rv1c,
           w1c, lb1c, g2a, bb2a, rm2a, rv2a, w2a, lb2a, g2b, bb2b, rm2b,
           rv2b, w2b, lb2b, g2c, bb2c, rm2c, rv2c, w2c, lb2c):
    w1a_f, b1a_f = _fold(g1a, bb1a, rm1a, rv1a, w1a, lb1a)
    w1b_f, b1b_f = _fold(g1b, bb1b, rm1b, rv1b, w1b, lb1b)
    w1c_f, b1c_f = _fold(g1c, bb1c, rm1c, rv1c, w1c, lb1c)
    w2a_f, b2a_f = _fold(g2a, bb2a, rm2a, rv2a, w2a, lb2a)
    w2b_f, b2b_f = _fold(g2b, bb2b, rm2b, rv2b, w2b, lb2b)
    w2c_f, b2c_f = _fold(g2c, bb2c, rm2c, rv2c, w2c, lb2c)
    w1ax, w1ae = w1a_f[:FX], w1a_f[FX:]
    w2ax, w2ag = w2a_f[:FX], w2a_f[FX:]

    ep = EPAD - EE
    row = jnp.concatenate(
        [edge_index[0].astype(jnp.int32), jnp.zeros((ep,), jnp.int32)])
    col = jnp.concatenate(
        [edge_index[1].astype(jnp.int32), jnp.full((ep,), NN, jnp.int32)])
    ea_pad = jnp.pad(edge_attr, ((0, ep), (0, 0)))
    x_pad = jnp.pad(x, ((0, NP - NN), (0, 0)))

    p = _tc_pre(x_pad, w1ax, b1a_f)
    gathered = _sc_gather(p, row.reshape(NCHUNK, CH))
    z2 = _tc_edge(gathered, ea_pad, w1ae, w1b_f, b1b_f)
    s_part, c_part = _sc_scatter(z2, col.reshape(NCHUNK, CH))
    out = _tc_node(s_part, c_part, x_pad, w1c_f, b1c_f, w2ax, w2ag, b2a_f,
                   w2b_f, b2b_f, w2c_f, b2c_f)
    return out[:NN]


# speedup vs baseline: 1.9122x; 1.6203x over previous
import functools

import jax
import jax.numpy as jnp
from jax import lax
from jax.experimental import pallas as pl
from jax.experimental.pallas import tpu as pltpu
from jax.experimental.pallas import tpu_sc as plsc

NN = 10000
NP = 10240
EE = 320000
EPAD = 327680
FX = 128
FE = 16
FO = 128
EPS = 1e-5
SLOPE = 0.1

CH = 128
NCHUNK = EPAD // CH
TPS = NP // 16


def _lrelu(v):
    return jnp.where(v >= 0.0, v, SLOPE * v)


def _sc_gather(p_pad, row):
    info = plsc.get_sparse_core_info()
    nc, ns = info.num_cores, info.num_subcores
    nw = nc * ns
    steps = NCHUNK // nw
    nbuf = 2
    groups = steps // nbuf

    @functools.partial(
        pl.kernel,
        mesh=plsc.VectorSubcoreMesh(core_axis_name="c", subcore_axis_name="s"),
        out_type=jax.ShapeDtypeStruct((EPAD, FO), jnp.float32),
        scratch_types=[
            pltpu.VMEM((steps, CH), jnp.int32),
            pltpu.VMEM((nbuf, CH, FO), jnp.float32),
        ] + [pltpu.SemaphoreType.DMA] * (2 * nbuf),
    )
    def k(p_hbm, row_hbm, out_hbm, idx_all, rows_v, *sems):
        sem_g = sems[:nbuf]
        sem_w = sems[nbuf:]
        cid = lax.axis_index("c")
        sid = lax.axis_index("s")
        wid = sid * nc + cid
        c0 = wid * steps

        pltpu.sync_copy(row_hbm.at[pl.ds(c0, steps)], idx_all)

        def gstart(u, t):
            pltpu.async_copy(p_hbm.at[idx_all.at[t]], rows_v.at[u], sem_g[u])

        def gwait(u, t):
            pltpu.make_async_copy(
                p_hbm.at[idx_all.at[t]], rows_v.at[u], sem_g[u]).wait()

        def wstart(u, t):
            pltpu.async_copy(rows_v.at[u],
                             out_hbm.at[pl.ds((c0 + t) * CH, CH)], sem_w[u])

        def wwait(u):
            pltpu.make_async_copy(
                rows_v.at[u], out_hbm.at[pl.ds(0, CH)], sem_w[u]).wait()

        for u in range(nbuf):
            gstart(u, u)

        def body(j2, carry):
            for u in range(nbuf):
                t = nbuf * j2 + u
                gwait(u, t)
                wstart(u, t)
                wwait(u)
                gstart(u, t + nbuf)
            return carry

        lax.fori_loop(0, groups - 1, body, 0)
        for u in range(nbuf):
            t = nbuf * (groups - 1) + u
            gwait(u, t)
            wstart(u, t)
        for u in range(nbuf):
            wwait(u)

    return k(p_pad, row)


def _sc_scatter(z2, col):
    info = plsc.get_sparse_core_info()
    nc, ns = info.num_cores, info.num_subcores
    nw = nc * ns
    steps = NCHUNK // nw
    nbuf = 2
    groups = steps // nbuf

    @functools.partial(
        pl.kernel,
        mesh=plsc.VectorSubcoreMesh(core_axis_name="c", subcore_axis_name="s"),
        out_type=(
            jax.ShapeDtypeStruct((2, NP, FO), jnp.float32),
            jax.ShapeDtypeStruct((2, NP), jnp.float32),
        ),
        scratch_types=[
            pltpu.VMEM((steps, CH), jnp.int32),
            pltpu.VMEM((nbuf, CH, FO), jnp.float32),
            pltpu.VMEM((TPS,), jnp.float32),
            pltpu.VMEM((CH,), jnp.float32),
            pltpu.VMEM_SHARED((NP, FO), jnp.float32),
            pltpu.VMEM_SHARED((NP,), jnp.float32),
        ] + [pltpu.SemaphoreType.DMA] * (3 * nbuf),
    )
    def k(z2_hbm, col_hbm, sum_hbm, cnt_hbm, idx_all, rows_v, zcnt,
          ones_v, acc, acc_cnt, *sems):
        sem_z = sems[:nbuf]
        sem_s = sems[nbuf:2 * nbuf]
        sem_c = sems[2 * nbuf:]
        cid = lax.axis_index("c")
        sid = lax.axis_index("s")
        wid = sid * nc + cid
        zf = jnp.zeros((16,), jnp.float32)
        of = jnp.ones((16,), jnp.float32)

        def zero_blk(i, carry):
            rows_v[0, i // 8, pl.ds((i % 8) * 16, 16)] = zf
            return carry

        lax.fori_loop(0, CH * FO // 16, zero_blk, 0)

        def zero_cnt(i, carry):
            zcnt[pl.ds(i * 16, 16)] = zf
            return carry

        lax.fori_loop(0, TPS // 16, zero_cnt, 0)
        for i in range(CH // 16):
            ones_v[pl.ds(i * 16, 16)] = of

        for i in range(TPS // CH):
            pltpu.sync_copy(rows_v.at[0], acc.at[pl.ds(sid * TPS + i * CH, CH)])
        pltpu.sync_copy(zcnt, acc_cnt.at[pl.ds(sid * TPS, TPS)])

        c0 = wid * steps
        pltpu.sync_copy(col_hbm.at[pl.ds(c0, steps)], idx_all)
        plsc.subcore_barrier()

        def zstart(u, t):
            pltpu.async_copy(z2_hbm.at[pl.ds((c0 + t) * CH, CH)],
                             rows_v.at[u], sem_z[u])

        def zwait(u):
            pltpu.make_async_copy(
                z2_hbm.at[pl.ds(0, CH)], rows_v.at[u], sem_z[u]).wait()

        def sstart(u, t):
            pltpu.async_copy(rows_v.at[u], acc.at[idx_all.at[t]], sem_s[u],
                             add=True)
            pltpu.async_copy(ones_v, acc_cnt.at[idx_all.at[t]], sem_c[u],
                             add=True)

        def swait(u, t):
            pltpu.make_async_copy(
                rows_v.at[u], acc.at[idx_all.at[t]], sem_s[u]).wait()
            pltpu.make_async_copy(
                ones_v, acc_cnt.at[idx_all.at[t]], sem_c[u]).wait()

        for u in range(nbuf):
            zstart(u, u)

        def body(j2, carry):
            for u in range(nbuf):
                t = nbuf * j2 + u
                zwait(u)
                sstart(u, t)
                swait(u, t)
                zstart(u, t + nbuf)
            return carry

        lax.fori_loop(0, groups - 1, body, 0)
        for u in range(nbuf):
            t = nbuf * (groups - 1) + u
            zwait(u)
            sstart(u, t)
            swait(u, t)
        plsc.subcore_barrier()

        pltpu.sync_copy(acc.at[pl.ds(sid * TPS, TPS)],
                        sum_hbm.at[cid].at[pl.ds(sid * TPS, TPS)])
        pltpu.sync_copy(acc_cnt.at[pl.ds(sid * TPS, TPS)],
                        cnt_hbm.at[cid].at[pl.ds(sid * TPS, TPS)])

    return k(z2, col)


def _tc_pre(x_pad, wx, b):
    br = 1280

    def body(x_ref, w_ref, b_ref, o_ref):
        o_ref[...] = (
            jnp.dot(x_ref[...], w_ref[...], preferred_element_type=jnp.float32)
            + b_ref[...]
        )

    return pl.pallas_call(
        body,
        grid=(NP // br,),
        in_specs=[
            pl.BlockSpec((br, FX), lambda i: (i, 0)),
            pl.BlockSpec((FX, FO), lambda i: (0, 0)),
            pl.BlockSpec((1, FO), lambda i: (0, 0)),
        ],
        out_specs=pl.BlockSpec((br, FO), lambda i: (i, 0)),
        out_shape=jax.ShapeDtypeStruct((NP, FO), jnp.float32),
    )(x_pad, wx, b.reshape(1, FO))


def _tc_edge(gathered, ea, wae, wb, bbias):
    be = 4096

    def body(g_ref, e_ref, wae_ref, wb_ref, b_ref, o_ref):
        h = g_ref[...] + jnp.dot(
            e_ref[...], wae_ref[...], preferred_element_type=jnp.float32
        )
        h = _lrelu(h)
        h = jnp.dot(h, wb_ref[...], preferred_element_type=jnp.float32) + b_ref[...]
        o_ref[...] = _lrelu(h)

    return pl.pallas_call(
        body,
        grid=(EPAD // be,),
        in_specs=[
            pl.BlockSpec((be, FO), lambda i: (i, 0)),
            pl.BlockSpec((be, FE), lambda i: (i, 0)),
            pl.BlockSpec((FE, FO), lambda i: (0, 0)),
            pl.BlockSpec((FO, FO), lambda i: (0, 0)),
            pl.BlockSpec((1, FO), lambda i: (0, 0)),
        ],
        out_specs=pl.BlockSpec((be, FO), lambda i: (i, 0)),
        out_shape=jax.ShapeDtypeStruct((EPAD, FO), jnp.float32),
    )(gathered, ea, wae, wb, bbias.reshape(1, FO))


def _tc_node(s_part, c_part, x_pad, wc, bc, wax, wag, ba, wb, bb2, wc2, bc2):
    br = 1280
    c3 = c_part.reshape(2, NP, 1)

    def body(s_ref, c_ref, x_ref, wc_ref, bc_ref, wax_ref, wag_ref, ba_ref,
             wb_ref, bb_ref, wc2_ref, bc2_ref, o_ref):
        s = s_ref[0] + s_ref[1]
        cnt = c_ref[0] + c_ref[1]
        sums = (
            jnp.dot(s, wc_ref[...], preferred_element_type=jnp.float32)
            + cnt * bc_ref[...]
        )
        agg = sums / jnp.maximum(cnt, 1.0)
        h = (
            jnp.dot(x_ref[...], wax_ref[...], preferred_element_type=jnp.float32)
            + jnp.dot(agg, wag_ref[...], preferred_element_type=jnp.float32)
            + ba_ref[...]
        )
        h = _lrelu(h)
        h = jnp.dot(h, wb_ref[...], preferred_element_type=jnp.float32) + bb_ref[...]
        h = _lrelu(h)
        o_ref[...] = (
            jnp.dot(h, wc2_ref[...], preferred_element_type=jnp.float32)
            + bc2_ref[...]
        )

    return pl.pallas_call(
        body,
        grid=(NP // br,),
        in_specs=[
            pl.BlockSpec((2, br, FO), lambda i: (0, i, 0)),
            pl.BlockSpec((2, br, 1), lambda i: (0, i, 0)),
            pl.BlockSpec((br, FX), lambda i: (i, 0)),
            pl.BlockSpec((FO, FO), lambda i: (0, 0)),
            pl.BlockSpec((1, FO), lambda i: (0, 0)),
            pl.BlockSpec((FX, FO), lambda i: (0, 0)),
            pl.BlockSpec((FO, FO), lambda i: (0, 0)),
            pl.BlockSpec((1, FO), lambda i: (0, 0)),
            pl.BlockSpec((FO, FO), lambda i: (0, 0)),
            pl.BlockSpec((1, FO), lambda i: (0, 0)),
            pl.BlockSpec((FO, FO), lambda i: (0, 0)),
            pl.BlockSpec((1, FO), lambda i: (0, 0)),
        ],
        out_specs=pl.BlockSpec((br, FO), lambda i: (i, 0)),
        out_shape=jax.ShapeDtypeStruct((NP, FO), jnp.float32),
    )(s_part, c3, x_pad, wc, bc.reshape(1, FO), wax, wag, ba.reshape(1, FO),
      wb, bb2.reshape(1, FO), wc2, bc2.reshape(1, FO))


def _fold(g, bb_, rm, rv, w, lb):
    s = g * lax.rsqrt(rv + EPS)
    t = bb_ - rm * s
    return w * s[:, None], t @ w + lb


def kernel(x, edge_index, edge_attr, u, batch, g1a, bb1a, rm1a, rv1a, w1a,
           lb1a, g1b, bb1b, rm1b, rv1b, w1b, lb1b, g1c, bb1c, rm1c, rv1c,
           w1c, lb1c, g2a, bb2a, rm2a, rv2a, w2a, lb2a, g2b, bb2b, rm2b,
           rv2b, w2b, lb2b, g2c, bb2c, rm2c, rv2c, w2c, lb2c):
    w1a_f, b1a_f = _fold(g1a, bb1a, rm1a, rv1a, w1a, lb1a)
    w1b_f, b1b_f = _fold(g1b, bb1b, rm1b, rv1b, w1b, lb1b)
    w1c_f, b1c_f = _fold(g1c, bb1c, rm1c, rv1c, w1c, lb1c)
    w2a_f, b2a_f = _fold(g2a, bb2a, rm2a, rv2a, w2a, lb2a)
    w2b_f, b2b_f = _fold(g2b, bb2b, rm2b, rv2b, w2b, lb2b)
    w2c_f, b2c_f = _fold(g2c, bb2c, rm2c, rv2c, w2c, lb2c)
    w1ax, w1ae = w1a_f[:FX], w1a_f[FX:]
    w2ax, w2ag = w2a_f[:FX], w2a_f[FX:]

    ep = EPAD - EE
    pad_iota = jnp.arange(ep, dtype=jnp.int32)
    row = jnp.concatenate(
        [edge_index[0].astype(jnp.int32), pad_iota % NN])
    col = jnp.concatenate(
        [edge_index[1].astype(jnp.int32), NN + pad_iota % (NP - NN)])
    ea_pad = jnp.pad(edge_attr, ((0, ep), (0, 0)))
    x_pad = jnp.pad(x, ((0, NP - NN), (0, 0)))

    p = _tc_pre(x_pad, w1ax, b1a_f)
    gathered = _sc_gather(p, row.reshape(NCHUNK, CH))
    z2 = _tc_edge(gathered, ea_pad, w1ae, w1b_f, b1b_f)
    s_part, c_part = _sc_scatter(z2, col.reshape(NCHUNK, CH))
    out = _tc_node(s_part, c_part, x_pad, w1c_f, b1c_f, w2ax, w2ag, b2a_f,
                   w2b_f, b2b_f, w2c_f, b2c_f)
    return out[:NN]
